# baseline (speedup 1.0000x reference)
import functools

import jax
import jax.numpy as jnp
from jax import lax
from jax.experimental import pallas as pl
from jax.experimental.pallas import tpu as pltpu
from jax.experimental.pallas import tpu_sc as plsc

N = 10000
E = 320000
D = 128

NC = 2
NS = 16
L = 16

CHUNK = 128
CPT = 80
HALF = CPT // 2
EPT = CPT * CHUNK
E_PAD = NC * NS * EPT
N_PAD = 10240
ROWS_PT = N_PAD // NS


@functools.cache
def _make_sc_aggregate():
    mesh = plsc.VectorSubcoreMesh(
        core_axis_name="c", subcore_axis_name="s",
        num_cores=NC, num_subcores=NS,
    )
    return pl.kernel(
        _sc_aggregate_body,
        out_type=(
            jax.ShapeDtypeStruct((NC, N_PAD, D), jnp.float32),
            jax.ShapeDtypeStruct((NC, N_PAD), jnp.float32),
        ),
        mesh=mesh,
        scratch_types=[
            pltpu.VMEM((HALF, CHUNK), jnp.int32),
            pltpu.VMEM((HALF, CHUNK), jnp.int32),
            pltpu.VMEM((CHUNK, D), jnp.float32),
            pltpu.VMEM((CHUNK, D), jnp.float32),
            pltpu.VMEM((CHUNK,), jnp.float32),
            pltpu.VMEM((ROWS_PT,), jnp.float32),
            pltpu.VMEM_SHARED((N_PAD, D), jnp.float32),
            pltpu.VMEM_SHARED((N_PAD,), jnp.float32),
            pltpu.SemaphoreType.DMA,
            pltpu.SemaphoreType.DMA,
            pltpu.SemaphoreType.DMA,
            pltpu.SemaphoreType.DMA,
        ],
    )


def _sc_aggregate_body(x_hbm, src_hbm, dst_hbm, agg_out, cnt_out,
                       src_v, dst_v, rows_a, rows_b, ones_v, zcnt_v,
                       agg_sh, cnt_sh, sem_a, sem_b, sem_c, sem_d):
    c = lax.axis_index("c")
    s = lax.axis_index("s")
    w = c * NS + s

    pltpu.async_copy(src_hbm.at[w, pl.ds(0, HALF)], src_v, sem_d)
    pltpu.async_copy(dst_hbm.at[w, pl.ds(0, HALF)], dst_v, sem_d)

    zeros16 = jnp.zeros((L,), jnp.float32)
    ones16 = jnp.ones((L,), jnp.float32)

    for j in range(CHUNK // L):
        ones_v[pl.ds(j * L, L)] = ones16

    def _zero_row(i, _):
        for j in range(D // L):
            rows_a[i, pl.ds(j * L, L)] = zeros16
        return 0

    lax.fori_loop(0, CHUNK, _zero_row, 0)

    def _zero_cnt(i, _):
        zcnt_v[pl.ds(i * L, L)] = zeros16
        return 0

    lax.fori_loop(0, ROWS_PT // L, _zero_cnt, 0)

    base = s * ROWS_PT
    for k in range(ROWS_PT // CHUNK):
        pltpu.async_copy(rows_a, agg_sh.at[pl.ds(base + k * CHUNK, CHUNK)],
                         sem_a)
    pltpu.async_copy(zcnt_v, cnt_sh.at[pl.ds(base, ROWS_PT)], sem_a)
    for k in range(ROWS_PT // CHUNK):
        pltpu.make_async_copy(
            rows_a, agg_sh.at[pl.ds(base + k * CHUNK, CHUNK)], sem_a).wait()
    pltpu.make_async_copy(zcnt_v, cnt_sh.at[pl.ds(base, ROWS_PT)], sem_a).wait()
    plsc.subcore_barrier()

    for p in range(2):
        pltpu.make_async_copy(src_hbm.at[w, pl.ds(0, HALF)], src_v, sem_d).wait()
        pltpu.make_async_copy(dst_hbm.at[w, pl.ds(0, HALF)], dst_v, sem_d).wait()
        pltpu.async_copy(x_hbm.at[src_v.at[0]], rows_a, sem_a)

        def _cnt(i, _):
            pltpu.async_copy(ones_v, cnt_sh.at[dst_v.at[i]], sem_c, add=True)
            return 0

        lax.fori_loop(0, HALF, _cnt, 0)

        def _pair(i, _):
            j0 = 2 * i
            j1 = j0 + 1
            j2 = j0 + 2
            pltpu.async_copy(x_hbm.at[src_v.at[j1]], rows_b, sem_b)
            pltpu.make_async_copy(x_hbm.at[src_v.at[j0]], rows_a, sem_a).wait()
            pltpu.sync_copy(rows_a, agg_sh.at[dst_v.at[j0]], add=True)

            @pl.when(j2 < HALF)
            def _():
                pltpu.async_copy(x_hbm.at[src_v.at[j2]], rows_a, sem_a)

            pltpu.make_async_copy(x_hbm.at[src_v.at[j1]], rows_b, sem_b).wait()
            pltpu.sync_copy(rows_b, agg_sh.at[dst_v.at[j1]], add=True)
            return 0

        lax.fori_loop(0, HALF // 2, _pair, 0)

        def _drain(i, _):
            pltpu.make_async_copy(ones_v, cnt_sh.at[dst_v.at[i]], sem_c).wait()
            return 0

        lax.fori_loop(0, HALF, _drain, 0)

        if p == 0:
            pltpu.async_copy(src_hbm.at[w, pl.ds(HALF, HALF)], src_v, sem_d)
            pltpu.async_copy(dst_hbm.at[w, pl.ds(HALF, HALF)], dst_v, sem_d)
    plsc.subcore_barrier()

    pltpu.async_copy(agg_sh.at[pl.ds(base, ROWS_PT)],
                     agg_out.at[c, pl.ds(base, ROWS_PT)], sem_a)
    pltpu.async_copy(cnt_sh.at[pl.ds(base, ROWS_PT)],
                     cnt_out.at[c, pl.ds(base, ROWS_PT)], sem_b)
    pltpu.make_async_copy(agg_sh.at[pl.ds(base, ROWS_PT)],
                          agg_out.at[c, pl.ds(base, ROWS_PT)], sem_a).wait()
    pltpu.make_async_copy(cnt_sh.at[pl.ds(base, ROWS_PT)],
                          cnt_out.at[c, pl.ds(base, ROWS_PT)], sem_b).wait()


BLK = 1024


def _tc_finish_body(agg_ref, cnt_ref, x_ref, wl_ref, wr_ref, bl_ref,
                    lnw_ref, lnb_ref, o_ref):
    a = agg_ref[0] + agg_ref[1]
    c16 = cnt_ref[0] + cnt_ref[1]

    rows = lax.broadcasted_iota(jnp.int32, (BLK, BLK // D), 0)
    cols = lax.broadcasted_iota(jnp.int32, (BLK, BLK // D), 1)
    sel = (cols == rows // D).astype(jnp.float32)
    t = jnp.dot(sel, c16, preferred_element_type=jnp.float32)
    rmod = lax.broadcasted_iota(jnp.int32, (BLK, D), 0) % D
    lane = lax.broadcasted_iota(jnp.int32, (BLK, D), 1)
    cnt = jnp.sum(jnp.where(lane == rmod, t, 0.0), axis=1, keepdims=True)

    mean = a / jnp.clip(cnt, 1.0, None)
    dn = (((1,), (1,)), ((), ()))
    out = (lax.dot_general(mean, wl_ref[...], dn,
                           preferred_element_type=jnp.float32)
           + lax.dot_general(x_ref[...], wr_ref[...], dn,
                             preferred_element_type=jnp.float32)
           + bl_ref[...])
    mu = jnp.mean(out, axis=-1, keepdims=True)
    var = jnp.mean((out - mu) ** 2, axis=-1, keepdims=True)
    out = (out - mu) * lax.rsqrt(var + 1e-5) * lnw_ref[...] + lnb_ref[...]
    o_ref[...] = jnp.maximum(out, 0.0)


def _tc_finish(agg, cnt_resh, x, wl, wr, bl2, lnw2, lnb2):
    grid = N_PAD // BLK
    return pl.pallas_call(
        _tc_finish_body,
        grid=(grid,),
        in_specs=[
            pl.BlockSpec((NC, BLK, D), lambda i: (0, i, 0)),
            pl.BlockSpec((NC, BLK // D, D), lambda i: (0, i, 0)),
            pl.BlockSpec((BLK, D), lambda i: (i, 0)),
            pl.BlockSpec((D, D), lambda i: (0, 0)),
            pl.BlockSpec((D, D), lambda i: (0, 0)),
            pl.BlockSpec((1, D), lambda i: (0, 0)),
            pl.BlockSpec((1, D), lambda i: (0, 0)),
            pl.BlockSpec((1, D), lambda i: (0, 0)),
        ],
        out_specs=pl.BlockSpec((BLK, D), lambda i: (i, 0)),
        out_shape=jax.ShapeDtypeStruct((N, D), jnp.float32),
    )(agg, cnt_resh, x, wl, wr, bl2, lnw2, lnb2)


def kernel(x, edge_index, W_l, b_l, W_r, ln_w, ln_b):
    src = edge_index[0]
    dst = edge_index[1]
    pad = E_PAD - E
    pad_src = (jnp.arange(pad, dtype=jnp.int32) * 8) % N
    pad_dst = N + (jnp.arange(pad, dtype=jnp.int32) % (N_PAD - N))
    src_t = jnp.concatenate([src, pad_src]).reshape(NC * NS, CPT, CHUNK)
    dst_t = jnp.concatenate([dst, pad_dst]).reshape(NC * NS, CPT, CHUNK)

    agg, cnt = _make_sc_aggregate()(x, src_t, dst_t)

    cnt_resh = cnt.reshape(NC, N_PAD // D, D)
    return _tc_finish(agg, cnt_resh, x, W_l, W_r,
                      b_l.reshape(1, D), ln_w.reshape(1, D), ln_b.reshape(1, D))


# speedup vs baseline: 1.0140x; 1.0140x over previous
import functools

import jax
import jax.numpy as jnp
from jax import lax
from jax.experimental import pallas as pl
from jax.experimental.pallas import tpu as pltpu
from jax.experimental.pallas import tpu_sc as plsc

N = 10000
E = 320000
D = 128

NC = 2
NS = 16
L = 16

CHUNK = 128
CPT = 80
HALF = CPT // 2
EPT = CPT * CHUNK
E_PAD = NC * NS * EPT
N_PAD = 10240
ROWS_PT = N_PAD // NS


@functools.cache
def _make_sc_aggregate():
    mesh = plsc.VectorSubcoreMesh(
        core_axis_name="c", subcore_axis_name="s",
        num_cores=NC, num_subcores=NS,
    )
    return pl.kernel(
        _sc_aggregate_body,
        out_type=(
            jax.ShapeDtypeStruct((NC, N_PAD, D), jnp.float32),
            jax.ShapeDtypeStruct((NC, N_PAD), jnp.float32),
        ),
        mesh=mesh,
        scratch_types=[
            pltpu.VMEM((HALF, CHUNK), jnp.int32),
            pltpu.VMEM((HALF, CHUNK), jnp.int32),
            pltpu.VMEM((CHUNK, D), jnp.float32),
            pltpu.VMEM((CHUNK, D), jnp.float32),
            pltpu.VMEM((CHUNK,), jnp.float32),
            pltpu.VMEM((ROWS_PT,), jnp.float32),
            pltpu.VMEM_SHARED((N_PAD, D), jnp.float32),
            pltpu.VMEM_SHARED((N_PAD,), jnp.float32),
            pltpu.SemaphoreType.DMA,
            pltpu.SemaphoreType.DMA,
            pltpu.SemaphoreType.DMA,
            pltpu.SemaphoreType.DMA,
        ],
    )


def _sc_aggregate_body(x_hbm, src_hbm, dst_hbm, agg_out, cnt_out,
                       src_v, dst_v, rows_a, rows_b, ones_v, zcnt_v,
                       agg_sh, cnt_sh, sem_a, sem_b, sem_c, sem_d):
    c = lax.axis_index("c")
    s = lax.axis_index("s")
    w = c * NS + s

    pltpu.async_copy(src_hbm.at[w, pl.ds(0, HALF)], src_v, sem_d)
    pltpu.async_copy(dst_hbm.at[w, pl.ds(0, HALF)], dst_v, sem_d)

    zeros16 = jnp.zeros((L,), jnp.float32)
    ones16 = jnp.ones((L,), jnp.float32)

    for j in range(CHUNK // L):
        ones_v[pl.ds(j * L, L)] = ones16

    def _zero_row(i, _):
        for j in range(D // L):
            rows_a[i, pl.ds(j * L, L)] = zeros16
        return 0

    lax.fori_loop(0, CHUNK, _zero_row, 0)

    def _zero_cnt(i, _):
        zcnt_v[pl.ds(i * L, L)] = zeros16
        return 0

    lax.fori_loop(0, ROWS_PT // L, _zero_cnt, 0)

    base = s * ROWS_PT
    for k in range(ROWS_PT // CHUNK):
        pltpu.async_copy(rows_a, agg_sh.at[pl.ds(base + k * CHUNK, CHUNK)],
                         sem_a)
    pltpu.async_copy(zcnt_v, cnt_sh.at[pl.ds(base, ROWS_PT)], sem_a)
    for k in range(ROWS_PT // CHUNK):
        pltpu.make_async_copy(
            rows_a, agg_sh.at[pl.ds(base + k * CHUNK, CHUNK)], sem_a).wait()
    pltpu.make_async_copy(zcnt_v, cnt_sh.at[pl.ds(base, ROWS_PT)], sem_a).wait()
    plsc.subcore_barrier()

    for p in range(2):
        pltpu.make_async_copy(src_hbm.at[w, pl.ds(0, HALF)], src_v, sem_d).wait()
        pltpu.make_async_copy(dst_hbm.at[w, pl.ds(0, HALF)], dst_v, sem_d).wait()
        pltpu.async_copy(x_hbm.at[src_v.at[0]], rows_a, sem_a)

        def _cnt(i, _):
            pltpu.async_copy(ones_v, cnt_sh.at[dst_v.at[i]], sem_c, add=True)
            return 0

        lax.fori_loop(0, HALF, _cnt, 0)

        def _pair(i, _):
            j0 = 2 * i
            j1 = j0 + 1
            j2 = j0 + 2
            pltpu.async_copy(x_hbm.at[src_v.at[j1]], rows_b, sem_b)
            pltpu.make_async_copy(x_hbm.at[src_v.at[j0]], rows_a, sem_a).wait()
            pltpu.sync_copy(rows_a, agg_sh.at[dst_v.at[j0]], add=True)

            @pl.when(j2 < HALF)
            def _():
                pltpu.async_copy(x_hbm.at[src_v.at[j2]], rows_a, sem_a)

            pltpu.make_async_copy(x_hbm.at[src_v.at[j1]], rows_b, sem_b).wait()
            pltpu.sync_copy(rows_b, agg_sh.at[dst_v.at[j1]], add=True)
            return 0

        lax.fori_loop(0, HALF // 2, _pair, 0)

        def _drain(i, _):
            pltpu.make_async_copy(ones_v, cnt_sh.at[dst_v.at[i]], sem_c).wait()
            return 0

        lax.fori_loop(0, HALF, _drain, 0)

        if p == 0:
            pltpu.async_copy(src_hbm.at[w, pl.ds(HALF, HALF)], src_v, sem_d)
            pltpu.async_copy(dst_hbm.at[w, pl.ds(HALF, HALF)], dst_v, sem_d)
    plsc.subcore_barrier()

    pltpu.sync_copy(agg_sh.at[pl.ds(base, ROWS_PT)],
                    agg_out.at[c, pl.ds(base, ROWS_PT)])
    pltpu.sync_copy(cnt_sh.at[pl.ds(base, ROWS_PT)],
                    cnt_out.at[c, pl.ds(base, ROWS_PT)])


BLK = 2048


def _tc_finish_body(agg_ref, cnt_ref, x_ref, wl_ref, wr_ref, bl_ref,
                    lnw_ref, lnb_ref, o_ref):
    a = agg_ref[0] + agg_ref[1]
    c16 = cnt_ref[0] + cnt_ref[1]

    rows = lax.broadcasted_iota(jnp.int32, (BLK, BLK // D), 0)
    cols = lax.broadcasted_iota(jnp.int32, (BLK, BLK // D), 1)
    sel = (cols == rows // D).astype(jnp.float32)
    t = jnp.dot(sel, c16, preferred_element_type=jnp.float32)
    rmod = lax.broadcasted_iota(jnp.int32, (BLK, D), 0) % D
    lane = lax.broadcasted_iota(jnp.int32, (BLK, D), 1)
    cnt = jnp.sum(jnp.where(lane == rmod, t, 0.0), axis=1, keepdims=True)

    mean = a / jnp.clip(cnt, 1.0, None)
    dn = (((1,), (1,)), ((), ()))
    out = (lax.dot_general(mean, wl_ref[...], dn,
                           preferred_element_type=jnp.float32)
           + lax.dot_general(x_ref[...], wr_ref[...], dn,
                             preferred_element_type=jnp.float32)
           + bl_ref[...])
    mu = jnp.mean(out, axis=-1, keepdims=True)
    var = jnp.mean((out - mu) ** 2, axis=-1, keepdims=True)
    out = (out - mu) * lax.rsqrt(var + 1e-5) * lnw_ref[...] + lnb_ref[...]
    o_ref[...] = jnp.maximum(out, 0.0)


def _tc_finish(agg, cnt_resh, x, wl, wr, bl2, lnw2, lnb2):
    grid = N_PAD // BLK
    return pl.pallas_call(
        _tc_finish_body,
        grid=(grid,),
        in_specs=[
            pl.BlockSpec((NC, BLK, D), lambda i: (0, i, 0)),
            pl.BlockSpec((NC, BLK // D, D), lambda i: (0, i, 0)),
            pl.BlockSpec((BLK, D), lambda i: (i, 0)),
            pl.BlockSpec((D, D), lambda i: (0, 0)),
            pl.BlockSpec((D, D), lambda i: (0, 0)),
            pl.BlockSpec((1, D), lambda i: (0, 0)),
            pl.BlockSpec((1, D), lambda i: (0, 0)),
            pl.BlockSpec((1, D), lambda i: (0, 0)),
        ],
        out_specs=pl.BlockSpec((BLK, D), lambda i: (i, 0)),
        out_shape=jax.ShapeDtypeStruct((N, D), jnp.float32),
    )(agg, cnt_resh, x, wl, wr, bl2, lnw2, lnb2)


def kernel(x, edge_index, W_l, b_l, W_r, ln_w, ln_b):
    src = edge_index[0]
    dst = edge_index[1]
    pad = E_PAD - E
    pad_src = (jnp.arange(pad, dtype=jnp.int32) * 8) % N
    pad_dst = N + (jnp.arange(pad, dtype=jnp.int32) % (N_PAD - N))
    src_t = jnp.concatenate([src, pad_src]).reshape(NC * NS, CPT, CHUNK)
    dst_t = jnp.concatenate([dst, pad_dst]).reshape(NC * NS, CPT, CHUNK)

    agg, cnt = _make_sc_aggregate()(x, src_t, dst_t)

    cnt_resh = cnt.reshape(NC, N_PAD // D, D)
    return _tc_finish(agg, cnt_resh, x, W_l, W_r,
                      b_l.reshape(1, D), ln_w.reshape(1, D), ln_b.reshape(1, D))
